# baseline (speedup 1.0000x reference)
import functools

import jax
import jax.numpy as jnp
from jax import lax
from jax.experimental import pallas as pl
from jax.experimental.pallas import tpu as pltpu
from jax.experimental.pallas import tpu_sc as plsc

_B = 16384
_D = 64
_NW = 32
_BPW = _B // _NW
_CH = 128
_NCH = _BPW // _CH


def _sc_gather(user, item, tug, tig, tum, tim):
    mesh = plsc.VectorSubcoreMesh(core_axis_name="c", subcore_axis_name="s")
    out_t = [jax.ShapeDtypeStruct((_B, _D), jnp.float32) for _ in range(4)]
    scratch = (
        [pltpu.VMEM((_CH,), jnp.int32) for _ in range(2 * _NCH)]
        + [pltpu.VMEM((_BPW, _D), jnp.float32) for _ in range(2)]
        + [pltpu.SemaphoreType.DMA]
    )

    @functools.partial(pl.kernel, mesh=mesh, out_type=out_t,
                       scratch_types=scratch,
                       compiler_params=pltpu.CompilerParams(
                           use_tc_tiling_on_sc=False))
    def body(user_h, item_h, tug_h, tig_h, tum_h, tim_h,
             o_gu, o_gi, o_mu, o_mi, *sc):
        idx = sc[:2 * _NCH]
        bufs = [sc[2 * _NCH], sc[2 * _NCH + 1]]
        gsem = sc[2 * _NCH + 2]
        c = lax.axis_index("c")
        s = lax.axis_index("s")
        base = (s * 2 + c) * _BPW
        for j in range(_NCH):
            pltpu.sync_copy(user_h.at[pl.ds(base + j * _CH, _CH)], idx[j])
            pltpu.sync_copy(item_h.at[pl.ds(base + j * _CH, _CH)],
                            idx[_NCH + j])

        tabs = [(tug_h, 0, o_gu), (tig_h, 1, o_gi),
                (tum_h, 0, o_mu), (tim_h, 1, o_mi)]

        def fire(t):
            tab, which, _ = tabs[t]
            hs = []
            for j in range(_NCH):
                hs.append(pltpu.async_copy(
                    tab.at[idx[which * _NCH + j]],
                    bufs[t % 2].at[pl.ds(j * _CH, _CH)], gsem))
            return hs

        handles = fire(0)
        for t in range(1, 4):
            for h in handles:
                h.wait()
            handles = fire(t)
            pltpu.sync_copy(bufs[(t - 1) % 2],
                            tabs[t - 1][2].at[pl.ds(base, _BPW)])
        for h in handles:
            h.wait()
        pltpu.sync_copy(bufs[1], tabs[3][2].at[pl.ds(base, _BPW)])

    return body(user, item, tug, tig, tum, tim)


def _tc_mlp(gu, gi, mu, mi, W1, b1, W2, b2, W3, b3, Wout, bout):
    bs = 2048
    grid = (_B // bs,)
    b1r = b1.reshape(1, -1)
    b2r = b2.reshape(1, -1)
    b3r = b3.reshape(1, -1)
    wa = Wout[:_D, 0].reshape(1, _D)
    wb = Wout[_D:, 0].reshape(1, -1)
    bor = bout.reshape(1, 1)

    def body(gu_r, gi_r, mu_r, mi_r, w1_r, b1_r, w2_r, b2_r, w3_r, b3_r,
             wa_r, wb_r, bo_r, out_r):
        h = jnp.dot(mu_r[...], w1_r[:_D, :],
                    preferred_element_type=jnp.float32)
        h = h + jnp.dot(mi_r[...], w1_r[_D:, :],
                        preferred_element_type=jnp.float32)
        h = jax.nn.relu(h + b1_r[...])
        h = jax.nn.relu(jnp.dot(h, w2_r[...],
                                preferred_element_type=jnp.float32) + b2_r[...])
        h = jax.nn.relu(jnp.dot(h, w3_r[...],
                                preferred_element_type=jnp.float32) + b3_r[...])
        g = gu_r[...] * gi_r[...]
        p = (jnp.sum(g * wa_r[...], axis=1, keepdims=True)
             + jnp.sum(h * wb_r[...], axis=1, keepdims=True) + bo_r[0, 0])
        out_r[...] = 1.0 / (1.0 + jnp.exp(-p))

    full = lambda a: pl.BlockSpec(a.shape, lambda i: (0,) * a.ndim)
    emb = pl.BlockSpec((bs, _D), lambda i: (i, 0))
    out = pl.pallas_call(
        body,
        grid=grid,
        in_specs=[emb, emb, emb, emb,
                  full(W1), full(b1r), full(W2), full(b2r),
                  full(W3), full(b3r), full(wa), full(wb), full(bor)],
        out_specs=pl.BlockSpec((bs, 1), lambda i: (i, 0)),
        out_shape=jax.ShapeDtypeStruct((_B, 1), jnp.float32),
    )(gu, gi, mu, mi, W1, b1r, W2, b2r, W3, b3r, wa, wb, bor)
    return out.reshape(_B)


def kernel(user, item, user_gmf_emb, item_gmf_emb, user_mlp_emb, item_mlp_emb,
           W1, b1, W2, b2, W3, b3, Wout, bout):
    gu, gi, mu, mi = _sc_gather(user, item, user_gmf_emb, item_gmf_emb,
                                user_mlp_emb, item_mlp_emb)
    return _tc_mlp(gu, gi, mu, mi, W1, b1, W2, b2, W3, b3, Wout, bout)


# speedup vs baseline: 1.5323x; 1.5323x over previous
import functools

import jax
import jax.numpy as jnp
from jax import lax
from jax.experimental import pallas as pl
from jax.experimental.pallas import tpu as pltpu
from jax.experimental.pallas import tpu_sc as plsc

_B = 16384
_D = 64
_NW = 32
_BPW = _B // _NW
_CH = 256
_NPH = _BPW // _CH


def _sc_gather(user, item, tug, tig, tum, tim):
    mesh = plsc.VectorSubcoreMesh(core_axis_name="c", subcore_axis_name="s")
    out_t = [jax.ShapeDtypeStruct((_B, 2 * _D), jnp.float32) for _ in range(2)]
    scratch = (
        [pltpu.VMEM((_BPW,), jnp.int32) for _ in range(2)]
        + [pltpu.VMEM((_CH, 2 * _D), jnp.float32) for _ in range(2)]
        + [pltpu.SemaphoreType.DMA, pltpu.SemaphoreType.DMA]
    )

    @functools.partial(pl.kernel, mesh=mesh, out_type=out_t,
                       scratch_types=scratch)
    def body(user_h, item_h, tug_h, tig_h, tum_h, tim_h,
             o_gmf, o_mlp, idxu, idxi, buf0, buf1, sem0, sem1):
        c = lax.axis_index("c")
        s = lax.axis_index("s")
        base = (s * 2 + c) * _BPW

        pltpu.sync_copy(user_h.at[pl.ds(base, _BPW)], idxu)
        pltpu.sync_copy(item_h.at[pl.ds(base, _BPW)], idxi)

        bufs = [buf0, buf1]
        sems = [sem0, sem1]

        def fire(tab_u, tab_i, off, buf, sem):
            def fbody(g, carry):
                vu = idxu[pl.ds(off + g * 16, 16)]
                vi = idxi[pl.ds(off + g * 16, 16)]
                for l in range(16):
                    j = g * 16 + l
                    pltpu.async_copy(tab_u.at[vu[l]],
                                     buf.at[j, pl.ds(0, _D)], sem)
                    pltpu.async_copy(tab_i.at[vi[l]],
                                     buf.at[j, pl.ds(_D, _D)], sem)
                return carry
            lax.fori_loop(0, _CH // 16, fbody, 0)

        def drain(buf, sem):
            pltpu.make_async_copy(tug_h.at[pl.ds(0, _CH)], buf, sem).wait()

        phases = [(tug_h, tig_h, o_gmf, 0), (tug_h, tig_h, o_gmf, _CH),
                  (tum_h, tim_h, o_mlp, 0), (tum_h, tim_h, o_mlp, _CH)]
        fire(*phases[0][:2], phases[0][3], bufs[0], sems[0])
        for p in range(1, 4):
            tu, ti, _, off = phases[p]
            fire(tu, ti, off, bufs[p % 2], sems[p % 2])
            pu, pi, pout, poff = phases[p - 1]
            drain(bufs[(p - 1) % 2], sems[(p - 1) % 2])
            pltpu.sync_copy(bufs[(p - 1) % 2],
                            pout.at[pl.ds(base + poff, _CH)])
        drain(bufs[1], sems[1])
        pltpu.sync_copy(bufs[1], phases[3][2].at[pl.ds(base + _CH, _CH)])

    return body(user, item, tug, tig, tum, tim)


def _tc_mlp(gmf2, mlp2, W1, b1, W2, b2, W3, b3, Wout, bout):
    bs = 2048
    grid = (_B // bs,)
    b1r = b1.reshape(1, -1)
    b2r = b2.reshape(1, -1)
    b3r = b3.reshape(1, -1)
    wa = Wout[:_D, 0].reshape(1, _D)
    wb = Wout[_D:, 0].reshape(1, -1)
    bor = bout.reshape(1, 1)

    def body(g_r, m_r, w1_r, b1_r, w2_r, b2_r, w3_r, b3_r,
             wa_r, wb_r, bo_r, out_r):
        h = jnp.dot(m_r[...], w1_r[...], preferred_element_type=jnp.float32)
        h = jax.nn.relu(h + b1_r[...])
        h = jax.nn.relu(jnp.dot(h, w2_r[...],
                                preferred_element_type=jnp.float32) + b2_r[...])
        h = jax.nn.relu(jnp.dot(h, w3_r[...],
                                preferred_element_type=jnp.float32) + b3_r[...])
        g = g_r[:, :_D] * g_r[:, _D:]
        p = (jnp.sum(g * wa_r[...], axis=1, keepdims=True)
             + jnp.sum(h * wb_r[...], axis=1, keepdims=True) + bo_r[0, 0])
        out_r[...] = 1.0 / (1.0 + jnp.exp(-p))

    full = lambda a: pl.BlockSpec(a.shape, lambda i: (0,) * a.ndim)
    emb = pl.BlockSpec((bs, 2 * _D), lambda i: (i, 0))
    out = pl.pallas_call(
        body,
        grid=grid,
        in_specs=[emb, emb,
                  full(W1), full(b1r), full(W2), full(b2r),
                  full(W3), full(b3r), full(wa), full(wb), full(bor)],
        out_specs=pl.BlockSpec((bs, 1), lambda i: (i, 0)),
        out_shape=jax.ShapeDtypeStruct((_B, 1), jnp.float32),
    )(gmf2, mlp2, W1, b1r, W2, b2r, W3, b3r, wa, wb, bor)
    return out.reshape(_B)


def kernel(user, item, user_gmf_emb, item_gmf_emb, user_mlp_emb, item_mlp_emb,
           W1, b1, W2, b2, W3, b3, Wout, bout):
    gmf2, mlp2 = _sc_gather(user, item, user_gmf_emb, item_gmf_emb,
                            user_mlp_emb, item_mlp_emb)
    return _tc_mlp(gmf2, mlp2, W1, b1, W2, b2, W3, b3, Wout, bout)
